# baseline (speedup 1.0000x reference)
import functools

import jax
import jax.numpy as jnp
from jax import lax
from jax.experimental import pallas as pl
from jax.experimental.pallas import tpu as pltpu
from jax.experimental.pallas import tpu_sc as plsc

VOCAB = 1000000
EMB_DIM = 64
PAD_DIM = 128
BATCH = 4096
SEQ = 200

N_IDX = BATCH * SEQ
NUM_WORKERS = 32
PER_WORKER = N_IDX // NUM_WORKERS
CHUNK = 320
NBUF = 2
STEPS = PER_WORKER // CHUNK
ROUNDS = STEPS // NBUF


def _make_gather():
    mesh = plsc.VectorSubcoreMesh(core_axis_name="c", subcore_axis_name="s")

    @functools.partial(
        pl.kernel,
        mesh=mesh,
        out_type=jax.ShapeDtypeStruct((N_IDX, PAD_DIM), jnp.float32),
        scratch_types=[
            pltpu.VMEM((PER_WORKER,), jnp.int32),
            pltpu.VMEM((NBUF, CHUNK, PAD_DIM), jnp.float32),
            pltpu.SemaphoreType.DMA,
            pltpu.SemaphoreType.DMA,
        ],
    )
    def gather_kernel(idx_hbm, table_hbm, out_hbm, idx_v, rows_v, sem_g, sem_w):
        wid = lax.axis_index("s") * 2 + lax.axis_index("c")
        wbase = wid * PER_WORKER

        pltpu.sync_copy(idx_hbm.at[pl.ds(wbase, PER_WORKER)], idx_v)

        def start_gather(t, b):
            pltpu.async_copy(
                table_hbm.at[idx_v.at[pl.ds(t * CHUNK, CHUNK)]],
                rows_v.at[b],
                sem_g,
            )

        def wait_gather(b):
            pltpu.make_async_copy(
                table_hbm.at[idx_v.at[pl.ds(0, CHUNK)]], rows_v.at[b], sem_g
            ).wait()

        def start_write(t, b):
            pltpu.async_copy(
                rows_v.at[b],
                out_hbm.at[pl.ds(wbase + t * CHUNK, CHUNK)],
                sem_w,
            )

        def wait_write(b):
            pltpu.make_async_copy(
                rows_v.at[b], out_hbm.at[pl.ds(wbase, CHUNK)], sem_w
            ).wait()

        for b in range(NBUF):
            start_gather(b, b)

        def round_body(k, _):
            for b in range(NBUF):
                wait_gather(b)
                start_write(k * NBUF + b, b)
            for b in range(NBUF):
                @pl.when(k < ROUNDS - 1)
                def _():
                    wait_write(b)
                    start_gather((k + 1) * NBUF + b, b)
            return ()

        lax.fori_loop(0, ROUNDS, round_body, (), unroll=False)

        for b in range(NBUF):
            wait_write(b)

    return gather_kernel


_gather = _make_gather()

_PREP_BLKV = 512


def _prep_body(tin, tout):
    x = tin[...]
    xz = jnp.concatenate([x, jnp.zeros((EMB_DIM, _PREP_BLKV), jnp.float32)], axis=0)
    tout[...] = jnp.transpose(xz)


def _tc_prep(tab_t):
    return pl.pallas_call(
        _prep_body,
        grid=(pl.cdiv(VOCAB, _PREP_BLKV),),
        in_specs=[pl.BlockSpec((EMB_DIM, _PREP_BLKV), lambda i: (0, i))],
        out_specs=pl.BlockSpec((_PREP_BLKV, PAD_DIM), lambda i: (i, 0)),
        out_shape=jax.ShapeDtypeStruct((VOCAB, PAD_DIM), jnp.float32),
    )(tab_t)


_OUT_BLKB = 512


_OUT_BLKS = 8


def _out_body(tin, tout):
    x = tin[...]
    for si in range(_OUT_BLKS):
        tout[si] = jnp.transpose(x[:, si, :])[:EMB_DIM, :]


def _tc_out(rows3):
    return pl.pallas_call(
        _out_body,
        grid=(SEQ // _OUT_BLKS, BATCH // _OUT_BLKB),
        in_specs=[
            pl.BlockSpec((_OUT_BLKB, _OUT_BLKS, PAD_DIM), lambda s, b: (b, s, 0))
        ],
        out_specs=pl.BlockSpec(
            (_OUT_BLKS, EMB_DIM, _OUT_BLKB), lambda s, b: (s, 0, b)
        ),
        out_shape=jax.ShapeDtypeStruct((SEQ, EMB_DIM, BATCH), jnp.float32),
    )(rows3)


def kernel(input_ids, attention_mask, emb_weight):
    tab128 = _tc_prep(emb_weight.T)
    flat_ids = input_ids.reshape(N_IDX)
    rows = _gather(flat_ids, tab128)
    outp = _tc_out(rows.reshape(BATCH, SEQ, PAD_DIM))
    out = jnp.transpose(outp, (2, 0, 1))
    return out, attention_mask


# speedup vs baseline: 1.2496x; 1.2496x over previous
import functools

import jax
import jax.numpy as jnp
from jax import lax
from jax.experimental import pallas as pl
from jax.experimental.pallas import tpu as pltpu
from jax.experimental.pallas import tpu_sc as plsc

VOCAB = 1000000
EMB_DIM = 64
PAD_DIM = 128
BATCH = 4096
SEQ = 200

N_IDX = BATCH * SEQ
NUM_WORKERS = 32
PER_WORKER = N_IDX // NUM_WORKERS
CHUNK = 320
NBUF = 2
STEPS = PER_WORKER // CHUNK
ROUNDS = STEPS // NBUF


def _make_gather():
    mesh = plsc.VectorSubcoreMesh(core_axis_name="c", subcore_axis_name="s")

    @functools.partial(
        pl.kernel,
        mesh=mesh,
        out_type=jax.ShapeDtypeStruct((N_IDX, PAD_DIM), jnp.float32),
        scratch_types=[
            pltpu.VMEM((PER_WORKER,), jnp.int32),
            pltpu.VMEM((NBUF, CHUNK, PAD_DIM), jnp.float32),
            pltpu.SemaphoreType.DMA,
            pltpu.SemaphoreType.DMA,
        ],
    )
    def gather_kernel(idx_hbm, table_hbm, out_hbm, idx_v, rows_v, sem_g, sem_w):
        wid = lax.axis_index("s") * 2 + lax.axis_index("c")
        wbase = wid * PER_WORKER

        pltpu.sync_copy(idx_hbm.at[pl.ds(wbase, PER_WORKER)], idx_v)

        def start_gather(t, b):
            pltpu.async_copy(
                table_hbm.at[idx_v.at[pl.ds(t * CHUNK, CHUNK)]],
                rows_v.at[b],
                sem_g,
            )

        def wait_gather(b):
            pltpu.make_async_copy(
                table_hbm.at[idx_v.at[pl.ds(0, CHUNK)]], rows_v.at[b], sem_g
            ).wait()

        def start_write(t, b):
            pltpu.async_copy(
                rows_v.at[b],
                out_hbm.at[pl.ds(wbase + t * CHUNK, CHUNK)],
                sem_w,
            )

        def wait_write(b):
            pltpu.make_async_copy(
                rows_v.at[b], out_hbm.at[pl.ds(wbase, CHUNK)], sem_w
            ).wait()

        for b in range(NBUF):
            start_gather(b, b)

        def round_body(k, _):
            for b in range(NBUF):
                wait_gather(b)
                start_write(k * NBUF + b, b)
            for b in range(NBUF):
                @pl.when(k < ROUNDS - 1)
                def _():
                    wait_write(b)
                    start_gather((k + 1) * NBUF + b, b)
            return ()

        lax.fori_loop(0, ROUNDS, round_body, (), unroll=False)

        for b in range(NBUF):
            wait_write(b)

    return gather_kernel


_gather = _make_gather()

_PREP_BLKV = 2048


def _prep_body(tin, tout):
    x = tin[...]
    sel = jnp.concatenate(
        [jnp.eye(EMB_DIM, dtype=jnp.float32),
         jnp.zeros((EMB_DIM, PAD_DIM - EMB_DIM), jnp.float32)], axis=1)
    tout[...] = jax.lax.dot_general(
        x, sel, (((0,), (0,)), ((), ())),
        precision=jax.lax.Precision.HIGHEST,
        preferred_element_type=jnp.float32,
    )


def _tc_prep(tab_t):
    return pl.pallas_call(
        _prep_body,
        grid=(pl.cdiv(VOCAB, _PREP_BLKV),),
        in_specs=[pl.BlockSpec((EMB_DIM, _PREP_BLKV), lambda i: (0, i))],
        out_specs=pl.BlockSpec((_PREP_BLKV, PAD_DIM), lambda i: (i, 0)),
        out_shape=jax.ShapeDtypeStruct((VOCAB, PAD_DIM), jnp.float32),
    )(tab_t)


_OUT_BLKB = 512


_OUT_BLKS = 8


def _out_body(tin, tout):
    x = tin[...]
    sel = jnp.concatenate(
        [jnp.eye(EMB_DIM, dtype=jnp.float32),
         jnp.zeros((EMB_DIM, PAD_DIM - EMB_DIM), jnp.float32)], axis=1)
    for si in range(_OUT_BLKS):
        tout[si] = jax.lax.dot_general(
            sel, x[:, si, :], (((1,), (1,)), ((), ())),
            precision=jax.lax.Precision.HIGHEST,
            preferred_element_type=jnp.float32,
        )


def _tc_out(rows3):
    return pl.pallas_call(
        _out_body,
        grid=(SEQ // _OUT_BLKS, BATCH // _OUT_BLKB),
        in_specs=[
            pl.BlockSpec((_OUT_BLKB, _OUT_BLKS, PAD_DIM), lambda s, b: (b, s, 0))
        ],
        out_specs=pl.BlockSpec(
            (_OUT_BLKS, EMB_DIM, _OUT_BLKB), lambda s, b: (s, 0, b)
        ),
        out_shape=jax.ShapeDtypeStruct((SEQ, EMB_DIM, BATCH), jnp.float32),
    )(rows3)


def kernel(input_ids, attention_mask, emb_weight):
    tab128 = _tc_prep(emb_weight.T)
    flat_ids = input_ids.reshape(N_IDX)
    rows = _gather(flat_ids, tab128)
    outp = _tc_out(rows.reshape(BATCH, SEQ, PAD_DIM))
    out = jnp.transpose(outp, (2, 0, 1))
    return out, attention_mask


# speedup vs baseline: 1.9074x; 1.5263x over previous
import functools

import jax
import jax.numpy as jnp
from jax import lax
from jax.experimental import pallas as pl
from jax.experimental.pallas import tpu as pltpu
from jax.experimental.pallas import tpu_sc as plsc

VOCAB = 1000000
EMB_DIM = 64
PAD_DIM = 128
BATCH = 4096
SEQ = 200

N_IDX = BATCH * SEQ
NUM_WORKERS = 32
PER_WORKER = N_IDX // NUM_WORKERS
CHUNK = 512
NBUF = 2
STEPS = PER_WORKER // CHUNK
ROUNDS = STEPS // NBUF


def _make_gather():
    mesh = plsc.VectorSubcoreMesh(core_axis_name="c", subcore_axis_name="s")

    @functools.partial(
        pl.kernel,
        mesh=mesh,
        out_type=jax.ShapeDtypeStruct((N_IDX, PAD_DIM), jnp.float32),
        scratch_types=[
            pltpu.VMEM((PER_WORKER,), jnp.int32),
            pltpu.VMEM((NBUF, CHUNK, EMB_DIM), jnp.float32),
            pltpu.SemaphoreType.DMA,
            pltpu.SemaphoreType.DMA,
        ],
        compiler_params=pltpu.CompilerParams(use_tc_tiling_on_sc=False),
    )
    def gather_kernel(idx_hbm, table_hbm, out_hbm, idx_v, rows_v, sem_g, sem_w):
        wid = lax.axis_index("s") * 2 + lax.axis_index("c")
        wbase = wid * PER_WORKER

        pltpu.sync_copy(idx_hbm.at[pl.ds(wbase, PER_WORKER)], idx_v)

        def start_gather(t, b):
            pltpu.async_copy(
                table_hbm.at[idx_v.at[pl.ds(t * CHUNK, CHUNK)]],
                rows_v.at[b],
                sem_g,
            )

        def wait_gather(b):
            pltpu.make_async_copy(
                table_hbm.at[idx_v.at[pl.ds(0, CHUNK)]], rows_v.at[b], sem_g
            ).wait()

        def start_write(t, b):
            pltpu.async_copy(
                rows_v.at[b],
                out_hbm.at[pl.ds(wbase + t * CHUNK, CHUNK), pl.ds(0, EMB_DIM)],
                sem_w,
            )

        def wait_write(b):
            pltpu.make_async_copy(
                rows_v.at[b],
                out_hbm.at[pl.ds(wbase, CHUNK), pl.ds(0, EMB_DIM)],
                sem_w,
            ).wait()

        for b in range(NBUF):
            start_gather(b, b)

        def round_body(k, _):
            for b in range(NBUF):
                wait_gather(b)
                start_write(k * NBUF + b, b)
            for b in range(NBUF):
                @pl.when(k < ROUNDS - 1)
                def _():
                    wait_write(b)
                    start_gather((k + 1) * NBUF + b, b)
            return ()

        lax.fori_loop(0, ROUNDS, round_body, (), unroll=False)

        for b in range(NBUF):
            wait_write(b)

    return gather_kernel


_gather = _make_gather()

_PREP_BLKV = 2048


def _prep_body(tin, tout):
    x = tin[...]
    sel = jnp.concatenate(
        [jnp.eye(EMB_DIM, dtype=jnp.float32),
         jnp.zeros((EMB_DIM, PAD_DIM - EMB_DIM), jnp.float32)], axis=1)
    tout[...] = jax.lax.dot_general(
        x, sel, (((0,), (0,)), ((), ())),
        precision=jax.lax.Precision.HIGHEST,
        preferred_element_type=jnp.float32,
    )


def _tc_prep(tab_t):
    return pl.pallas_call(
        _prep_body,
        grid=(pl.cdiv(VOCAB, _PREP_BLKV),),
        in_specs=[pl.BlockSpec((EMB_DIM, _PREP_BLKV), lambda i: (0, i))],
        out_specs=pl.BlockSpec((_PREP_BLKV, PAD_DIM), lambda i: (i, 0)),
        out_shape=jax.ShapeDtypeStruct((VOCAB, PAD_DIM), jnp.float32),
    )(tab_t)


def kernel(input_ids, attention_mask, emb_weight):
    tab128 = _tc_prep(emb_weight.T)
    tab2 = tab128.reshape(2 * VOCAB, EMB_DIM)
    ids2 = (input_ids.reshape(N_IDX) * 2).astype(jnp.int32)
    rows = _gather(ids2, tab2)
    out = rows.reshape(BATCH, SEQ, PAD_DIM)[:, :, :EMB_DIM]
    return out, attention_mask


# speedup vs baseline: 2.1159x; 1.1093x over previous
import functools

import jax
import jax.numpy as jnp
from jax import lax
from jax.experimental import pallas as pl
from jax.experimental.pallas import tpu as pltpu
from jax.experimental.pallas import tpu_sc as plsc

VOCAB = 1000000
EMB_DIM = 64
PAD_DIM = 128
BATCH = 4096
SEQ = 200

N_IDX = BATCH * SEQ
NUM_WORKERS = 32
PER_WORKER = N_IDX // NUM_WORKERS
CHUNK = 640
NBUF = 2
STEPS = PER_WORKER // CHUNK
ROUNDS = STEPS // NBUF


def _make_gather():
    mesh = plsc.VectorSubcoreMesh(core_axis_name="c", subcore_axis_name="s")

    @functools.partial(
        pl.kernel,
        mesh=mesh,
        out_type=jax.ShapeDtypeStruct((N_IDX, PAD_DIM), jnp.float32),
        scratch_types=[
            pltpu.VMEM((PER_WORKER,), jnp.int32),
            pltpu.VMEM((NBUF, CHUNK, EMB_DIM), jnp.float32),
            pltpu.SemaphoreType.DMA,
            pltpu.SemaphoreType.DMA,
        ],
        compiler_params=pltpu.CompilerParams(use_tc_tiling_on_sc=False),
    )
    def gather_kernel(idx_hbm, table_hbm, out_hbm, idx_v, rows_v, sem_g, sem_w):
        wid = lax.axis_index("s") * 2 + lax.axis_index("c")
        wbase = wid * PER_WORKER

        pltpu.sync_copy(idx_hbm.at[pl.ds(wbase, PER_WORKER)], idx_v)

        def start_gather(t, b):
            pltpu.async_copy(
                table_hbm.at[idx_v.at[pl.ds(t * CHUNK, CHUNK)]],
                rows_v.at[b],
                sem_g,
            )

        def wait_gather(b):
            pltpu.make_async_copy(
                table_hbm.at[idx_v.at[pl.ds(0, CHUNK)]], rows_v.at[b], sem_g
            ).wait()

        def start_write(t, b):
            pltpu.async_copy(
                rows_v.at[b],
                out_hbm.at[pl.ds(wbase + t * CHUNK, CHUNK), pl.ds(0, EMB_DIM)],
                sem_w,
            )

        def wait_write(b):
            pltpu.make_async_copy(
                rows_v.at[b],
                out_hbm.at[pl.ds(wbase, CHUNK), pl.ds(0, EMB_DIM)],
                sem_w,
            ).wait()

        for b in range(NBUF):
            start_gather(b, b)

        def round_body(k, _):
            for b in range(NBUF):
                wait_gather(b)
                start_write(k * NBUF + b, b)
            for b in range(NBUF):
                @pl.when(k < ROUNDS - 1)
                def _():
                    wait_write(b)
                    start_gather((k + 1) * NBUF + b, b)
            return ()

        lax.fori_loop(0, ROUNDS, round_body, (), unroll=False)

        for b in range(NBUF):
            wait_write(b)

    return gather_kernel


_gather = _make_gather()

_PREP_BLKV = 2048


def _prep_body(tin, tout):
    x = tin[...]
    sel = jnp.concatenate(
        [jnp.eye(EMB_DIM, dtype=jnp.float32),
         jnp.zeros((EMB_DIM, PAD_DIM - EMB_DIM), jnp.float32)], axis=1)
    x_hi = x.astype(jnp.bfloat16).astype(jnp.float32)
    x_lo = x - x_hi

    def tdot(a):
        return jax.lax.dot_general(
            a, sel, (((0,), (0,)), ((), ())),
            precision=jax.lax.Precision.DEFAULT,
            preferred_element_type=jnp.float32,
        )

    tout[...] = tdot(x_hi) + tdot(x_lo)


def _tc_prep(tab_t):
    return pl.pallas_call(
        _prep_body,
        grid=(pl.cdiv(VOCAB, _PREP_BLKV),),
        in_specs=[pl.BlockSpec((EMB_DIM, _PREP_BLKV), lambda i: (0, i))],
        out_specs=pl.BlockSpec((_PREP_BLKV, PAD_DIM), lambda i: (i, 0)),
        out_shape=jax.ShapeDtypeStruct((VOCAB, PAD_DIM), jnp.float32),
    )(tab_t)


def kernel(input_ids, attention_mask, emb_weight):
    tab128 = _tc_prep(emb_weight.T)
    tab2 = tab128.reshape(2 * VOCAB, EMB_DIM)
    ids2 = (input_ids.reshape(N_IDX) * 2).astype(jnp.int32)
    rows = _gather(ids2, tab2)
    out = rows.reshape(BATCH, SEQ, PAD_DIM)[:, :, :EMB_DIM]
    return out, attention_mask


# speedup vs baseline: 2.1226x; 1.0031x over previous
import functools

import jax
import jax.numpy as jnp
from jax import lax
from jax.experimental import pallas as pl
from jax.experimental.pallas import tpu as pltpu
from jax.experimental.pallas import tpu_sc as plsc

VOCAB = 1000000
EMB_DIM = 64
PAD_DIM = 128
BATCH = 4096
SEQ = 200

N_IDX = BATCH * SEQ
NUM_WORKERS = 32
PER_WORKER = N_IDX // NUM_WORKERS
CHUNK = 640
NBUF = 2
STEPS = PER_WORKER // CHUNK
ROUNDS = STEPS // NBUF


def _make_gather():
    mesh = plsc.VectorSubcoreMesh(core_axis_name="c", subcore_axis_name="s")

    @functools.partial(
        pl.kernel,
        mesh=mesh,
        out_type=jax.ShapeDtypeStruct((N_IDX, PAD_DIM), jnp.float32),
        scratch_types=[
            pltpu.VMEM((PER_WORKER,), jnp.int32),
            pltpu.VMEM((NBUF, CHUNK, EMB_DIM), jnp.float32),
            pltpu.SemaphoreType.DMA,
            pltpu.SemaphoreType.DMA,
        ],
        compiler_params=pltpu.CompilerParams(use_tc_tiling_on_sc=False),
    )
    def gather_kernel(idx_hbm, table_hbm, out_hbm, idx_v, rows_v, sem_g, sem_w):
        wid = lax.axis_index("s") * 2 + lax.axis_index("c")
        wbase = wid * PER_WORKER

        pltpu.sync_copy(idx_hbm.at[pl.ds(wbase, PER_WORKER)], idx_v)

        def dbl(i, _):
            sl = pl.ds(i * 16, 16)
            idx_v[sl] = idx_v[sl] * 2
            return ()

        lax.fori_loop(0, PER_WORKER // 16, dbl, (), unroll=8)

        def start_gather(t, b):
            pltpu.async_copy(
                table_hbm.at[idx_v.at[pl.ds(t * CHUNK, CHUNK)]],
                rows_v.at[b],
                sem_g,
            )

        def wait_gather(b):
            pltpu.make_async_copy(
                table_hbm.at[idx_v.at[pl.ds(0, CHUNK)]], rows_v.at[b], sem_g
            ).wait()

        def start_write(t, b):
            pltpu.async_copy(
                rows_v.at[b],
                out_hbm.at[pl.ds(wbase + t * CHUNK, CHUNK), pl.ds(0, EMB_DIM)],
                sem_w,
            )

        def wait_write(b):
            pltpu.make_async_copy(
                rows_v.at[b],
                out_hbm.at[pl.ds(wbase, CHUNK), pl.ds(0, EMB_DIM)],
                sem_w,
            ).wait()

        for b in range(NBUF):
            start_gather(b, b)

        def round_body(k, _):
            for b in range(NBUF):
                wait_gather(b)
                start_write(k * NBUF + b, b)
            for b in range(NBUF):
                @pl.when(k < ROUNDS - 1)
                def _():
                    wait_write(b)
                    start_gather((k + 1) * NBUF + b, b)
            return ()

        lax.fori_loop(0, ROUNDS, round_body, (), unroll=False)

        for b in range(NBUF):
            wait_write(b)

    return gather_kernel


_gather = _make_gather()

_PREP_BLKV = 2048


def _prep_body(tin, tout):
    x = tin[...]
    sel = jnp.concatenate(
        [jnp.eye(EMB_DIM, dtype=jnp.float32),
         jnp.zeros((EMB_DIM, PAD_DIM - EMB_DIM), jnp.float32)], axis=1)
    x_hi = x.astype(jnp.bfloat16).astype(jnp.float32)
    x_lo = x - x_hi

    def tdot(a):
        return jax.lax.dot_general(
            a, sel, (((0,), (0,)), ((), ())),
            precision=jax.lax.Precision.DEFAULT,
            preferred_element_type=jnp.float32,
        )

    tout[...] = tdot(x_hi) + tdot(x_lo)


def _tc_prep(tab_t):
    return pl.pallas_call(
        _prep_body,
        grid=(pl.cdiv(VOCAB, _PREP_BLKV),),
        in_specs=[pl.BlockSpec((EMB_DIM, _PREP_BLKV), lambda i: (0, i))],
        out_specs=pl.BlockSpec((_PREP_BLKV, PAD_DIM), lambda i: (i, 0)),
        out_shape=jax.ShapeDtypeStruct((VOCAB, PAD_DIM), jnp.float32),
    )(tab_t)


def kernel(input_ids, attention_mask, emb_weight):
    tab2 = _tc_prep(emb_weight.T).reshape(2 * VOCAB, EMB_DIM)
    flat_ids = input_ids.reshape(N_IDX)
    rows = _gather(flat_ids, tab2)
    out = rows.reshape(BATCH, SEQ, PAD_DIM)[:, :, :EMB_DIM]
    return out, attention_mask


# speedup vs baseline: 2.7568x; 1.2988x over previous
import functools

import jax
import jax.numpy as jnp
from jax import lax
from jax.experimental import pallas as pl
from jax.experimental.pallas import tpu as pltpu
from jax.experimental.pallas import tpu_sc as plsc

VOCAB = 1000000
EMB_DIM = 64
PAD_DIM = 128
BATCH = 4096
SEQ = 200

N_IDX = BATCH * SEQ
NUM_WORKERS = 32
PER_WORKER = N_IDX // NUM_WORKERS
CHUNK = 640
NBUF = 2
STEPS = PER_WORKER // CHUNK
ROUNDS = STEPS // NBUF


def _make_gather():
    mesh = plsc.VectorSubcoreMesh(core_axis_name="c", subcore_axis_name="s")

    @functools.partial(
        pl.kernel,
        mesh=mesh,
        out_type=jax.ShapeDtypeStruct((N_IDX, PAD_DIM), jnp.float32),
        scratch_types=[
            pltpu.VMEM((PER_WORKER,), jnp.int32),
            pltpu.VMEM((NBUF, CHUNK, EMB_DIM), jnp.float32),
            pltpu.SemaphoreType.DMA,
            pltpu.SemaphoreType.DMA,
        ],
        compiler_params=pltpu.CompilerParams(use_tc_tiling_on_sc=False),
    )
    def gather_kernel(idx_hbm, table_hbm, out_hbm, idx_v, rows_v, sem_g, sem_w):
        wid = lax.axis_index("s") * 2 + lax.axis_index("c")
        wbase = wid * PER_WORKER

        pltpu.sync_copy(idx_hbm.at[pl.ds(wbase, PER_WORKER)], idx_v)

        def dbl(i, _):
            sl = pl.ds(i * 16, 16)
            idx_v[sl] = idx_v[sl] * 2
            return ()

        lax.fori_loop(0, PER_WORKER // 16, dbl, (), unroll=8)

        def start_gather(t, b):
            pltpu.async_copy(
                table_hbm.at[idx_v.at[pl.ds(t * CHUNK, CHUNK)]],
                rows_v.at[b],
                sem_g,
            )

        def wait_gather(b):
            pltpu.make_async_copy(
                table_hbm.at[idx_v.at[pl.ds(0, CHUNK)]], rows_v.at[b], sem_g
            ).wait()

        def start_write(t, b):
            pltpu.async_copy(
                rows_v.at[b],
                out_hbm.at[pl.ds(wbase + t * CHUNK, CHUNK), pl.ds(0, EMB_DIM)],
                sem_w,
            )

        def wait_write(b):
            pltpu.make_async_copy(
                rows_v.at[b],
                out_hbm.at[pl.ds(wbase, CHUNK), pl.ds(0, EMB_DIM)],
                sem_w,
            ).wait()

        for b in range(NBUF):
            start_gather(b, b)

        def round_body(k, _):
            for b in range(NBUF):
                wait_gather(b)
                start_write(k * NBUF + b, b)
            for b in range(NBUF):
                @pl.when(k < ROUNDS - 1)
                def _():
                    wait_write(b)
                    start_gather((k + 1) * NBUF + b, b)
            return ()

        lax.fori_loop(0, ROUNDS, round_body, (), unroll=False)

        for b in range(NBUF):
            wait_write(b)

    return gather_kernel


_gather = _make_gather()

_PREP_BLKV = 8192


def _prep_body(tin, tout):
    x = tin[...]
    sel = jnp.concatenate(
        [jnp.eye(EMB_DIM, dtype=jnp.float32),
         jnp.zeros((EMB_DIM, PAD_DIM - EMB_DIM), jnp.float32)], axis=1)
    x_hi = x.astype(jnp.bfloat16).astype(jnp.float32)
    x_lo = x - x_hi

    def tdot(a):
        return jax.lax.dot_general(
            a, sel, (((0,), (0,)), ((), ())),
            precision=jax.lax.Precision.DEFAULT,
            preferred_element_type=jnp.float32,
        )

    tout[...] = tdot(x_hi) + tdot(x_lo)


def _tc_prep(tab_t):
    return pl.pallas_call(
        _prep_body,
        grid=(pl.cdiv(VOCAB, _PREP_BLKV),),
        in_specs=[pl.BlockSpec((EMB_DIM, _PREP_BLKV), lambda i: (0, i))],
        out_specs=pl.BlockSpec((_PREP_BLKV, PAD_DIM), lambda i: (i, 0)),
        out_shape=jax.ShapeDtypeStruct((VOCAB, PAD_DIM), jnp.float32),
    )(tab_t)


def kernel(input_ids, attention_mask, emb_weight):
    tab2 = _tc_prep(emb_weight.T).reshape(2 * VOCAB, EMB_DIM)
    flat_ids = input_ids.reshape(N_IDX)
    rows = _gather(flat_ids, tab2)
    out = rows.reshape(BATCH, SEQ, PAD_DIM)[:, :, :EMB_DIM]
    return out, attention_mask


# speedup vs baseline: 2.9047x; 1.0536x over previous
import functools

import jax
import jax.numpy as jnp
from jax import lax
from jax.experimental import pallas as pl
from jax.experimental.pallas import tpu as pltpu
from jax.experimental.pallas import tpu_sc as plsc

VOCAB = 1000000
EMB_DIM = 64
PAD_DIM = 128
BATCH = 4096
SEQ = 200

N_IDX = BATCH * SEQ
NUM_WORKERS = 32
PER_WORKER = N_IDX // NUM_WORKERS
CHUNK = 640
NBUF = 2
STEPS = PER_WORKER // CHUNK
ROUNDS = STEPS // NBUF


def _make_gather():
    mesh = plsc.VectorSubcoreMesh(core_axis_name="c", subcore_axis_name="s")

    @functools.partial(
        pl.kernel,
        mesh=mesh,
        out_type=jax.ShapeDtypeStruct((N_IDX, PAD_DIM), jnp.float32),
        scratch_types=[
            pltpu.VMEM((PER_WORKER,), jnp.int32),
            pltpu.VMEM((NBUF, CHUNK, EMB_DIM), jnp.float32),
            pltpu.SemaphoreType.DMA,
            pltpu.SemaphoreType.DMA,
        ],
        compiler_params=pltpu.CompilerParams(use_tc_tiling_on_sc=False),
    )
    def gather_kernel(idx_hbm, table_hbm, out_hbm, idx_v, rows_v, sem_g, sem_w):
        wid = lax.axis_index("s") * 2 + lax.axis_index("c")
        wbase = wid * PER_WORKER

        pltpu.sync_copy(idx_hbm.at[pl.ds(wbase, PER_WORKER)], idx_v)

        def dbl(i, _):
            sl = pl.ds(i * 16, 16)
            idx_v[sl] = idx_v[sl] * 2
            return ()

        lax.fori_loop(0, PER_WORKER // 16, dbl, (), unroll=8)

        def start_gather(t, b):
            pltpu.async_copy(
                table_hbm.at[idx_v.at[pl.ds(t * CHUNK, CHUNK)]],
                rows_v.at[b],
                sem_g,
            )

        def wait_gather(b):
            pltpu.make_async_copy(
                table_hbm.at[idx_v.at[pl.ds(0, CHUNK)]], rows_v.at[b], sem_g
            ).wait()

        def start_write(t, b):
            pltpu.async_copy(
                rows_v.at[b],
                out_hbm.at[pl.ds(wbase + t * CHUNK, CHUNK), pl.ds(0, EMB_DIM)],
                sem_w,
            )

        def wait_write(b):
            pltpu.make_async_copy(
                rows_v.at[b],
                out_hbm.at[pl.ds(wbase, CHUNK), pl.ds(0, EMB_DIM)],
                sem_w,
            ).wait()

        for b in range(NBUF):
            start_gather(b, b)

        def round_body(k, _):
            for b in range(NBUF):
                wait_gather(b)
                start_write(k * NBUF + b, b)
            for b in range(NBUF):
                @pl.when(k < ROUNDS - 1)
                def _():
                    wait_write(b)
                    start_gather((k + 1) * NBUF + b, b)
            return ()

        lax.fori_loop(0, ROUNDS, round_body, (), unroll=False)

        for b in range(NBUF):
            wait_write(b)

    return gather_kernel


_gather = _make_gather()

_PREP_BLKV = 16384


def _prep_body(tin, tout):
    x = tin[...]
    sel = jnp.concatenate(
        [jnp.eye(EMB_DIM, dtype=jnp.float32),
         jnp.zeros((EMB_DIM, PAD_DIM - EMB_DIM), jnp.float32)], axis=1)
    x_hi = x.astype(jnp.bfloat16).astype(jnp.float32)
    x_lo = x - x_hi

    def tdot(a):
        return jax.lax.dot_general(
            a, sel, (((0,), (0,)), ((), ())),
            precision=jax.lax.Precision.DEFAULT,
            preferred_element_type=jnp.float32,
        )

    tout[...] = tdot(x_hi) + tdot(x_lo)


def _tc_prep(tab_t):
    return pl.pallas_call(
        _prep_body,
        grid=(pl.cdiv(VOCAB, _PREP_BLKV),),
        in_specs=[pl.BlockSpec((EMB_DIM, _PREP_BLKV), lambda i: (0, i))],
        out_specs=pl.BlockSpec((_PREP_BLKV, PAD_DIM), lambda i: (i, 0)),
        out_shape=jax.ShapeDtypeStruct((VOCAB, PAD_DIM), jnp.float32),
    )(tab_t)


def kernel(input_ids, attention_mask, emb_weight):
    tab2 = _tc_prep(emb_weight.T).reshape(2 * VOCAB, EMB_DIM)
    flat_ids = input_ids.reshape(N_IDX)
    rows = _gather(flat_ids, tab2)
    out = rows.reshape(BATCH, SEQ, PAD_DIM)[:, :, :EMB_DIM]
    return out, attention_mask
